# baseline (speedup 1.0000x reference)
import functools

import jax
import jax.numpy as jnp
from jax import lax
from jax.experimental import pallas as pl
from jax.experimental.pallas import tpu as pltpu
from jax.experimental.pallas import tpu_sc as plsc

NV = 10242
NV_PREV = 2562
B = 8
C_IN = 128
C_OUT = 128
NNZ = 7
CB = C_IN * B
NW = 32
VPT = 336
NV_PAD = NW * VPT
VCHUNK = 8
EPC = VCHUNK * NNZ
NCHUNK = VPT // VCHUNK
IDX_PT = VPT * NNZ
TBL_ROWS = 2816
LANES = 16
JSTEPS = CB // LANES


def _bcast_lane(vec, lane):
    idx = jnp.full((LANES, 1), lane, jnp.int32)
    dn = lax.GatherDimensionNumbers(
        offset_dims=(), collapsed_slice_dims=(0,), start_index_map=(0,))
    return lax.gather(vec, idx, dn, slice_sizes=(1,),
                      mode=lax.GatherScatterMode.PROMISE_IN_BOUNDS)


def _sc_spmm(tbl, cols3, vals3):
    mesh = plsc.VectorSubcoreMesh(core_axis_name="c", subcore_axis_name="s")

    @functools.partial(
        pl.kernel,
        mesh=mesh,
        out_type=[jax.ShapeDtypeStruct((NV_PAD, B, C_IN), jnp.float32)] * 3,
        scratch_types=[
            pltpu.VMEM((IDX_PT,), jnp.int32),
            pltpu.VMEM((2, 4 * LANES), jnp.float32),
            pltpu.VMEM((2, EPC, B, C_IN), jnp.float32),
            pltpu.VMEM((VCHUNK, B, C_IN), jnp.float32),
            pltpu.SemaphoreType.DMA,
            pltpu.SemaphoreType.DMA,
            pltpu.SemaphoreType.DMA,
        ],
    )
    def body(cols_h, vals_h, tbl_h, g0, g1, g2, idx_v, vbuf, gbuf, obuf,
             sem0, sem1, osem):
        wid = lax.axis_index("s") * 2 + lax.axis_index("c")
        gsem = (sem0, sem1)

        def gather(m, c, par):
            off = pl.multiple_of(c * EPC, 8)
            voff = pl.multiple_of((m * NW + wid) * IDX_PT + c * EPC, 8)
            return (
                pltpu.make_async_copy(
                    tbl_h.at[idx_v.at[pl.ds(off, EPC)]], gbuf.at[par],
                    gsem[par]),
                pltpu.make_async_copy(
                    vals_h.at[pl.ds(voff, EPC)],
                    vbuf.at[par, pl.ds(0, EPC)], gsem[par]),
            )

        for m, out_h in enumerate((g0, g1, g2)):
            coff = pl.multiple_of((m * NW + wid) * IDX_PT, 8)
            pltpu.sync_copy(cols_h.at[pl.ds(coff, IDX_PT)], idx_v)
            for d in gather(m, 0, 0):
                d.start()

            def chunk_pair(c2, carry, m=m, out_h=out_h):
                for par in range(2):
                    c = c2 * 2 + par

                    @pl.when(c + 1 < NCHUNK)
                    def _():
                        for d in gather(m, c + 1, (par + 1) % 2):
                            d.start()

                    for d in gather(m, c, par):
                        d.wait()

                    @pl.when(c > 0)
                    def _():
                        pltpu.make_async_copy(
                            obuf, out_h.at[pl.ds(0, VCHUNK)], osem).wait()

                    vc = [vbuf[par, pl.ds(i * LANES, LANES)]
                          for i in range(4)]
                    for v in range(VCHUNK):
                        vv = [
                            _bcast_lane(vc[(v * NNZ + k) // LANES],
                                        (v * NNZ + k) % LANES)
                            for k in range(NNZ)
                        ]

                        @plsc.parallel_loop(0, JSTEPS, unroll=8)
                        def jbody(j, par=par, v=v, vv=vv):
                            s = j // 8
                            sl = pl.ds(
                                pl.multiple_of((j % 8) * LANES, LANES),
                                LANES)
                            acc = vv[0] * gbuf[par, v * NNZ, s, sl]
                            for k in range(1, NNZ):
                                acc = (acc
                                       + vv[k] * gbuf[par, v * NNZ + k, s, sl])
                            obuf[v, s, sl] = acc

                    row = pl.multiple_of(wid * VPT + c * VCHUNK, 8)
                    pltpu.make_async_copy(
                        obuf, out_h.at[pl.ds(row, VCHUNK)], osem).start()
                return carry

            lax.fori_loop(0, NCHUNK // 2, chunk_pair, 0)
            pltpu.make_async_copy(
                obuf, out_h.at[pl.ds(0, VCHUNK)], osem).wait()

    return body(cols3, vals3, tbl)


def _tc_mix(tbl_r, gl, ge, gn, coeffs, bias2):
    VB = (NV + 127) // 128
    TB = TBL_ROWS // 128

    def tc_body(tbl_ref, gl_ref, ge_ref, gn_ref, w_ref, b_ref, out_ref):
        w = w_ref[...]
        cn = (((0,), (1,)), ((), ()))

        def dt(wm, x):
            return lax.dot_general(wm, x, cn,
                                   precision=lax.Precision.DEFAULT,
                                   preferred_element_type=jnp.float32)

        for b in range(B):
            acc = dt(w[0], tbl_ref[:, b, :])
            acc += dt(w[1], gl_ref[:, b, :])
            acc += dt(w[2], ge_ref[:, b, :])
            acc += dt(w[3], gn_ref[:, b, :])
            out_ref[b, :, :] = acc + b_ref[0, :][:, None]

    return pl.pallas_call(
        tc_body,
        grid=(VB,),
        in_specs=[
            pl.BlockSpec((128, B, 128),
                         lambda vb: (jnp.minimum(vb, TB - 1), 0, 0)),
            pl.BlockSpec((128, B, 128), lambda vb: (vb, 0, 0)),
            pl.BlockSpec((128, B, 128), lambda vb: (vb, 0, 0)),
            pl.BlockSpec((128, B, 128), lambda vb: (vb, 0, 0)),
            pl.BlockSpec((4, 128, 128), lambda vb: (0, 0, 0)),
            pl.BlockSpec((1, 128), lambda vb: (0, 0)),
        ],
        out_specs=pl.BlockSpec((B, 128, 128), lambda vb: (0, 0, vb)),
        out_shape=jax.ShapeDtypeStruct((B, C_OUT, NV), jnp.float32),
    )(tbl_r, gl, ge, gn, coeffs, bias2)


def kernel(input, coeffs, bias, L_rows, L_cols, L_vals, EW_rows, EW_cols,
           EW_vals, NS_rows, NS_cols, NS_vals):
    x = input.astype(jnp.float32)
    tblr = jnp.transpose(x, (2, 0, 1))
    tbl = jnp.concatenate(
        [tblr, jnp.ones((TBL_ROWS - NV_PREV, B, C_IN), jnp.float32)], axis=0)

    pad = NV_PAD * NNZ - NV * NNZ
    n_ones = TBL_ROWS - NV_PREV
    pad_tail = NV_PREV + (jnp.arange(pad, dtype=jnp.int32) % n_ones)

    def prep(cols, vals):
        c = jnp.where(cols < NV_PREV, cols,
                      NV_PREV + ((cols - NV_PREV) % n_ones))
        return (jnp.concatenate([c, pad_tail]).reshape(NW, IDX_PT),
                jnp.pad(vals, (0, pad)).reshape(NW, IDX_PT))

    cL, vL = prep(L_cols, L_vals)
    cE, vE = prep(EW_cols, EW_vals)
    cN, vN = prep(NS_cols, NS_vals)
    cols3 = jnp.stack([cL, cE, cN]).reshape(-1)
    vals3 = jnp.stack([vL, vE, vN]).reshape(-1)

    g0, g1, g2 = _sc_spmm(tbl, cols3, vals3)

    return _tc_mix(tbl, g0, g1, g2, coeffs, bias.reshape(1, C_OUT))


# speedup vs baseline: 1.0176x; 1.0176x over previous
import functools

import jax
import jax.numpy as jnp
from jax import lax
from jax.experimental import pallas as pl
from jax.experimental.pallas import tpu as pltpu
from jax.experimental.pallas import tpu_sc as plsc

NV = 10242
NV_PREV = 2562
B = 8
C_IN = 128
C_OUT = 128
NNZ = 7
CB = C_IN * B
NW = 32
VPT = 336
NV_PAD = NW * VPT
VCHUNK = 8
EPC = VCHUNK * NNZ
NCHUNK = VPT // VCHUNK
IDX_PT = VPT * NNZ
TBL_ROWS = 2816
LANES = 16
JSTEPS = CB // LANES


def _bcast_lane(vec, lane):
    idx = jnp.full((LANES, 1), lane, jnp.int32)
    dn = lax.GatherDimensionNumbers(
        offset_dims=(), collapsed_slice_dims=(0,), start_index_map=(0,))
    return lax.gather(vec, idx, dn, slice_sizes=(1,),
                      mode=lax.GatherScatterMode.PROMISE_IN_BOUNDS)


def _sc_spmm(tbl, cols3, vals3):
    mesh = plsc.VectorSubcoreMesh(core_axis_name="c", subcore_axis_name="s")

    @functools.partial(
        pl.kernel,
        mesh=mesh,
        out_type=[jax.ShapeDtypeStruct((NV_PAD, B, C_IN), jnp.float32)] * 3,
        scratch_types=[
            pltpu.VMEM((IDX_PT,), jnp.int32),
            pltpu.VMEM((2, 4 * LANES), jnp.float32),
            pltpu.VMEM((2, EPC, B, C_IN), jnp.float32),
            pltpu.VMEM((VCHUNK, B, C_IN), jnp.float32),
            pltpu.SemaphoreType.DMA,
            pltpu.SemaphoreType.DMA,
            pltpu.SemaphoreType.DMA,
        ],
    )
    def body(cols_h, vals_h, tbl_h, g0, g1, g2, idx_v, vbuf, gbuf, obuf,
             sem0, sem1, osem):
        wid = lax.axis_index("s") * 2 + lax.axis_index("c")
        gsem = (sem0, sem1)

        def gather(m, c, par):
            off = pl.multiple_of(c * EPC, 8)
            voff = pl.multiple_of((m * NW + wid) * IDX_PT + c * EPC, 8)
            return (
                pltpu.make_async_copy(
                    tbl_h.at[idx_v.at[pl.ds(off, EPC)]], gbuf.at[par],
                    gsem[par]),
                pltpu.make_async_copy(
                    vals_h.at[pl.ds(voff, EPC)],
                    vbuf.at[par, pl.ds(0, EPC)], gsem[par]),
            )

        for m, out_h in enumerate((g0, g1, g2)):
            coff = pl.multiple_of((m * NW + wid) * IDX_PT, 8)
            pltpu.sync_copy(cols_h.at[pl.ds(coff, IDX_PT)], idx_v)
            for d in gather(m, 0, 0):
                d.start()

            def chunk_pair(c2, carry, m=m, out_h=out_h):
                for par in range(2):
                    c = c2 * 2 + par

                    @pl.when(c + 1 < NCHUNK)
                    def _():
                        for d in gather(m, c + 1, (par + 1) % 2):
                            d.start()

                    for d in gather(m, c, par):
                        d.wait()

                    @pl.when(c > 0)
                    def _():
                        pltpu.make_async_copy(
                            obuf, out_h.at[pl.ds(0, VCHUNK)], osem).wait()

                    vc = [vbuf[par, pl.ds(i * LANES, LANES)]
                          for i in range(4)]
                    for v in range(VCHUNK):
                        vv = [
                            _bcast_lane(vc[(v * NNZ + k) // LANES],
                                        (v * NNZ + k) % LANES)
                            for k in range(NNZ)
                        ]

                        @plsc.parallel_loop(0, JSTEPS, unroll=4)
                        def jbody(j, par=par, v=v, vv=vv):
                            s = j // 8
                            sl = pl.ds(
                                pl.multiple_of((j % 8) * LANES, LANES),
                                LANES)
                            acc = vv[0] * gbuf[par, v * NNZ, s, sl]
                            for k in range(1, NNZ):
                                acc = (acc
                                       + vv[k] * gbuf[par, v * NNZ + k, s, sl])
                            obuf[v, s, sl] = acc

                    row = pl.multiple_of(wid * VPT + c * VCHUNK, 8)
                    pltpu.make_async_copy(
                        obuf, out_h.at[pl.ds(row, VCHUNK)], osem).start()
                return carry

            lax.fori_loop(0, NCHUNK // 2, chunk_pair, 0)
            pltpu.make_async_copy(
                obuf, out_h.at[pl.ds(0, VCHUNK)], osem).wait()

    return body(cols3, vals3, tbl)


def _tc_mix(tbl_r, gl, ge, gn, coeffs, bias2):
    VB = (NV + 127) // 128
    TB = TBL_ROWS // 128

    def tc_body(tbl_ref, gl_ref, ge_ref, gn_ref, w_ref, b_ref, out_ref):
        w = w_ref[...]
        cn = (((0,), (1,)), ((), ()))

        def dt(wm, x):
            return lax.dot_general(wm, x, cn,
                                   precision=lax.Precision.DEFAULT,
                                   preferred_element_type=jnp.float32)

        for b in range(B):
            acc = dt(w[0], tbl_ref[:, b, :])
            acc += dt(w[1], gl_ref[:, b, :])
            acc += dt(w[2], ge_ref[:, b, :])
            acc += dt(w[3], gn_ref[:, b, :])
            out_ref[b, :, :] = acc + b_ref[0, :][:, None]

    return pl.pallas_call(
        tc_body,
        grid=(VB,),
        in_specs=[
            pl.BlockSpec((128, B, 128),
                         lambda vb: (jnp.minimum(vb, TB - 1), 0, 0)),
            pl.BlockSpec((128, B, 128), lambda vb: (vb, 0, 0)),
            pl.BlockSpec((128, B, 128), lambda vb: (vb, 0, 0)),
            pl.BlockSpec((128, B, 128), lambda vb: (vb, 0, 0)),
            pl.BlockSpec((4, 128, 128), lambda vb: (0, 0, 0)),
            pl.BlockSpec((1, 128), lambda vb: (0, 0)),
        ],
        out_specs=pl.BlockSpec((B, 128, 128), lambda vb: (0, 0, vb)),
        out_shape=jax.ShapeDtypeStruct((B, C_OUT, NV), jnp.float32),
    )(tbl_r, gl, ge, gn, coeffs, bias2)


def kernel(input, coeffs, bias, L_rows, L_cols, L_vals, EW_rows, EW_cols,
           EW_vals, NS_rows, NS_cols, NS_vals):
    x = input.astype(jnp.float32)
    tblr = jnp.transpose(x, (2, 0, 1))
    tbl = jnp.concatenate(
        [tblr, jnp.ones((TBL_ROWS - NV_PREV, B, C_IN), jnp.float32)], axis=0)

    pad = NV_PAD * NNZ - NV * NNZ
    n_ones = TBL_ROWS - NV_PREV
    pad_tail = NV_PREV + (jnp.arange(pad, dtype=jnp.int32) % n_ones)

    def prep(cols, vals):
        c = jnp.where(cols < NV_PREV, cols,
                      NV_PREV + ((cols - NV_PREV) % n_ones))
        return (jnp.concatenate([c, pad_tail]).reshape(NW, IDX_PT),
                jnp.pad(vals, (0, pad)).reshape(NW, IDX_PT))

    cL, vL = prep(L_cols, L_vals)
    cE, vE = prep(EW_cols, EW_vals)
    cN, vN = prep(NS_cols, NS_vals)
    cols3 = jnp.stack([cL, cE, cN]).reshape(-1)
    vals3 = jnp.stack([vL, vE, vN]).reshape(-1)

    g0, g1, g2 = _sc_spmm(tbl, cols3, vals3)

    return _tc_mix(tbl, g0, g1, g2, coeffs, bias.reshape(1, C_OUT))
